# baseline (speedup 1.0000x reference)
import functools
import math

import jax
import jax.numpy as jnp
from jax import lax
from jax.experimental import pallas as pl
from jax.experimental.pallas import tpu as pltpu
from jax.experimental.pallas import tpu_sc as plsc

BASE_GSD = 10.0

_NW = 32
_ROWS_PER_W = 3072
_CH = 192
_NCHUNK = _ROWS_PER_W // _CH
_HW_PER_CHUNK = _CH // 48


def _sc_body(tok, a1n, sp, months, mtab, out,
             in0, in1, out0, out1, a1_v, sp_v, mons_v, mrows_v,
             sg0, sg1, ss0, ss1, si):
    wid = lax.axis_index("s") * 2 + lax.axis_index("c")
    b = wid // 4
    q = wid % 4
    hw0 = q * 64
    row0 = b * (256 * 48) + hw0 * 48

    pltpu.sync_copy(a1n, a1_v)
    pltpu.sync_copy(sp.at[pl.ds(hw0, 64)], sp_v)
    pltpu.sync_copy(months.at[b], mons_v)
    pltpu.async_copy(mtab.at[mons_v], mrows_v, si).wait()

    for t in range(12):
        m0 = mrows_v[t, pl.ds(0, 16)]
        m1 = mrows_v[t, pl.ds(16, 16)]
        for c4 in range(4):
            a1_v[4 * t + c4, pl.ds(64, 16)] = m0
            a1_v[4 * t + c4, pl.ds(80, 16)] = m1

    ibufs = (in0, in1)
    obufs = (out0, out1)
    gsems = (sg0, sg1)
    ssems = (ss0, ss1)

    def g_copy(c, p):
        return pltpu.make_async_copy(
            tok.at[pl.ds(row0 + c * _CH, _CH)], ibufs[p], gsems[p])

    def s_copy(c, p):
        return pltpu.make_async_copy(
            obufs[p], out.at[pl.ds(row0 + c * _CH, _CH)], ssems[p])

    g_copy(0, 0).start()
    g_copy(1, 1).start()

    for c in range(_NCHUNK):
        p = c & 1
        g_copy(c, p).wait()
        if c >= 2:
            s_copy(c - 2, p).wait()
        bi, bo = ibufs[p], obufs[p]
        sregs = []
        for h in range(_HW_PER_CHUNK):
            hwrow = c * _HW_PER_CHUNK + h
            sregs.append((sp_v[hwrow, pl.ds(0, 16)],
                          sp_v[hwrow, pl.ds(16, 16)]))

        def rbody(r, _, bi=bi, bo=bo, sregs=sregs):
            a = [a1_v[r, pl.ds(16 * j, 16)] for j in range(6)]
            for h in range(_HW_PER_CHUNK):
                base = h * 48
                for j in range(6):
                    bo[base + r, pl.ds(16 * j, 16)] = (
                        bi[base + r, pl.ds(16 * j, 16)] + a[j])
                bo[base + r, pl.ds(96, 16)] = (
                    bi[base + r, pl.ds(96, 16)] + sregs[h][0])
                bo[base + r, pl.ds(112, 16)] = (
                    bi[base + r, pl.ds(112, 16)] + sregs[h][1])
            return 0

        lax.fori_loop(0, 48, rbody, 0)
        s_copy(c, p).start()
        if c + 2 < _NCHUNK:
            g_copy(c + 2, p).start()

    s_copy(_NCHUNK - 2, 0 if (_NCHUNK - 2) % 2 == 0 else 1).wait()
    s_copy(_NCHUNK - 1, 0 if (_NCHUNK - 1) % 2 == 0 else 1).wait()


def kernel(per_modality_input_tokens, timestamps, channel_embed, patch_size,
           input_res):
    x = per_modality_input_tokens
    b, h, w, t, cg, D = x.shape
    f32 = jnp.float32
    n_rows = b * h * w * t * cg
    xr = x.reshape(n_rows, D)

    om16 = 1.0 / (10000.0 ** (jnp.arange(16, dtype=f32) / 16.0))
    targ = jnp.arange(t, dtype=f32)[:, None] * om16[None, :]
    pos32 = jnp.concatenate([jnp.sin(targ), jnp.cos(targ)], axis=1)
    mang = jnp.arange(12, dtype=f32) / f32(12.0 / (2.0 * math.pi))
    mtab = jnp.concatenate([
        jnp.broadcast_to(jnp.sin(mang)[:, None], (12, 16)),
        jnp.broadcast_to(jnp.cos(mang)[:, None], (12, 16)),
        jnp.zeros((12, D - 32), f32),
    ], axis=1)
    gsd = (jnp.asarray(input_res).astype(f32)
           * jnp.asarray(patch_size).astype(f32) / BASE_GSD)
    om8 = 1.0 / (10000.0 ** (jnp.arange(8, dtype=f32) / 8.0))
    iv = jnp.arange(h, dtype=f32)[:, None].repeat(w, 1).reshape(-1) * gsd
    jv = jnp.arange(w, dtype=f32)[None, :].repeat(h, 0).reshape(-1) * gsd
    ai = iv[:, None] * om8[None, :]
    aj = jv[:, None] * om8[None, :]
    sp = jnp.concatenate(
        [jnp.sin(aj), jnp.cos(aj), jnp.sin(ai), jnp.cos(ai)], axis=1)

    ch48 = jnp.tile(channel_embed.astype(f32), (t, 1))
    pos48 = jnp.repeat(pos32, cg, axis=0)
    a1n = jnp.concatenate(
        [ch48, pos48, jnp.zeros((t * cg, 64), f32)], axis=1)

    months = jnp.zeros((b, 16), jnp.int32)
    months = months.at[:, :t].set(timestamps[:, 1, :].astype(jnp.int32))

    mesh = plsc.VectorSubcoreMesh(core_axis_name="c", subcore_axis_name="s")
    run = pl.kernel(
        _sc_body, mesh=mesh,
        out_type=jax.ShapeDtypeStruct((n_rows, D), f32),
        scratch_types=[
            pltpu.VMEM((_CH, D), f32),
            pltpu.VMEM((_CH, D), f32),
            pltpu.VMEM((_CH, D), f32),
            pltpu.VMEM((_CH, D), f32),
            pltpu.VMEM((t * cg, D), f32),
            pltpu.VMEM((64, 32), f32),
            pltpu.VMEM((16,), jnp.int32),
            pltpu.VMEM((16, D), f32),
            pltpu.SemaphoreType.DMA,
            pltpu.SemaphoreType.DMA,
            pltpu.SemaphoreType.DMA,
            pltpu.SemaphoreType.DMA,
            pltpu.SemaphoreType.DMA,
        ],
    )
    out = run(xr, a1n, sp, months, mtab)
    return out.reshape(b, h, w, t, cg, D)


# speedup vs baseline: 1.0108x; 1.0108x over previous
import math

import jax
import jax.numpy as jnp
from jax import lax
from jax.experimental import pallas as pl
from jax.experimental.pallas import tpu as pltpu
from jax.experimental.pallas import tpu_sc as plsc

BASE_GSD = 10.0
HW_BLK = 64


def _sc_table_body(a1n, months, mtab, out, a1_v, mons_v, mrows_v, si):
    wid = lax.axis_index("s") * 2 + lax.axis_index("c")
    nb = out.shape[0]

    @pl.when(wid < nb)
    def _():
        b = wid
        pltpu.sync_copy(a1n, a1_v)
        pltpu.sync_copy(months.at[b], mons_v)
        pltpu.async_copy(mtab.at[mons_v], mrows_v, si).wait()
        for t in range(12):
            m0 = mrows_v[t, pl.ds(0, 16)]
            m1 = mrows_v[t, pl.ds(16, 16)]
            for c4 in range(4):
                a1_v[4 * t + c4, pl.ds(64, 16)] = m0
                a1_v[4 * t + c4, pl.ds(80, 16)] = m1
        pltpu.sync_copy(a1_v, out.at[b])


def _tc_dense_body(gsd_ref, a1_ref, x_ref, o_ref):
    f32 = jnp.float32
    gsd = gsd_ref[0, 0]
    hwb = pl.program_id(1)
    a1 = a1_ref[0]

    hw = hwb * HW_BLK + lax.broadcasted_iota(jnp.int32, (HW_BLK, 8), 0)
    iv = (hw // 16).astype(f32) * gsd
    jv = (hw % 16).astype(f32) * gsd
    om8 = 1.0 / (10000.0 ** (
        lax.broadcasted_iota(jnp.int32, (HW_BLK, 8), 1).astype(f32) / 8.0))
    aj = jv * om8
    ai = iv * om8
    sp = jnp.concatenate([
        jnp.zeros((HW_BLK, 96), f32),
        jnp.sin(aj), jnp.cos(aj), jnp.sin(ai), jnp.cos(ai),
    ], axis=1)

    o_ref[0] = x_ref[0] + a1[None, :, :] + sp[:, None, :]


def kernel(per_modality_input_tokens, timestamps, channel_embed, patch_size,
           input_res):
    x = per_modality_input_tokens
    b, h, w, t, cg, D = x.shape
    f32 = jnp.float32
    xr = x.reshape(b, h * w, t * cg, D)

    om16 = 1.0 / (10000.0 ** (jnp.arange(16, dtype=f32) / 16.0))
    targ = jnp.arange(t, dtype=f32)[:, None] * om16[None, :]
    pos32 = jnp.concatenate([jnp.sin(targ), jnp.cos(targ)], axis=1)
    mang = jnp.arange(12, dtype=f32) / f32(12.0 / (2.0 * math.pi))
    mtab = jnp.concatenate([
        jnp.broadcast_to(jnp.sin(mang)[:, None], (12, 16)),
        jnp.broadcast_to(jnp.cos(mang)[:, None], (12, 16)),
        jnp.zeros((12, D - 32), f32),
    ], axis=1)

    ch48 = jnp.tile(channel_embed.astype(f32), (t, 1))
    pos48 = jnp.repeat(pos32, cg, axis=0)
    a1n = jnp.concatenate(
        [ch48, pos48, jnp.zeros((t * cg, 64), f32)], axis=1)

    months = jnp.zeros((b, 16), jnp.int32)
    months = months.at[:, :t].set(timestamps[:, 1, :].astype(jnp.int32))

    mesh = plsc.VectorSubcoreMesh(core_axis_name="c", subcore_axis_name="s")
    a1_full = pl.kernel(
        _sc_table_body, mesh=mesh,
        out_type=jax.ShapeDtypeStruct((b, t * cg, D), f32),
        scratch_types=[
            pltpu.VMEM((t * cg, D), f32),
            pltpu.VMEM((16,), jnp.int32),
            pltpu.VMEM((16, D), f32),
            pltpu.SemaphoreType.DMA,
        ],
    )(a1n, months, mtab)

    gsd = (jnp.asarray(input_res).astype(f32)
           * jnp.asarray(patch_size).astype(f32) / BASE_GSD).reshape(1, 1)
    out = pl.pallas_call(
        _tc_dense_body,
        grid=(b, (h * w) // HW_BLK),
        in_specs=[
            pl.BlockSpec(memory_space=pltpu.SMEM),
            pl.BlockSpec((1, t * cg, D), lambda bi, hi: (bi, 0, 0)),
            pl.BlockSpec((1, HW_BLK, t * cg, D), lambda bi, hi: (bi, hi, 0, 0)),
        ],
        out_specs=pl.BlockSpec((1, HW_BLK, t * cg, D),
                               lambda bi, hi: (bi, hi, 0, 0)),
        out_shape=jax.ShapeDtypeStruct(xr.shape, xr.dtype),
        compiler_params=pltpu.CompilerParams(
            dimension_semantics=("parallel", "parallel")),
    )(gsd, a1_full, xr)
    return out.reshape(b, h, w, t, cg, D)


# speedup vs baseline: 1.1504x; 1.1380x over previous
import math

import jax
import jax.numpy as jnp
from jax import lax
from jax.experimental import pallas as pl
from jax.experimental.pallas import tpu as pltpu
from jax.experimental.pallas import tpu_sc as plsc

BASE_GSD = 10.0
HW_BLK = 128


def _sc_table_body(a1n, months, mtab, out, a1_v, mons_v, mrows_v, si):
    wid = lax.axis_index("s") * 2 + lax.axis_index("c")
    nb = out.shape[0]

    @pl.when(wid < nb)
    def _():
        b = wid
        pltpu.sync_copy(a1n, a1_v)
        pltpu.sync_copy(months.at[b], mons_v)
        pltpu.async_copy(mtab.at[mons_v], mrows_v, si).wait()
        for t in range(12):
            m0 = mrows_v[t, pl.ds(0, 16)]
            m1 = mrows_v[t, pl.ds(16, 16)]
            for c4 in range(4):
                a1_v[4 * t + c4, pl.ds(64, 16)] = m0
                a1_v[4 * t + c4, pl.ds(80, 16)] = m1
        pltpu.sync_copy(a1_v, out.at[b])


def _tc_dense_body(gsd_ref, a1_ref, x_ref, o_ref):
    f32 = jnp.float32
    gsd = gsd_ref[0, 0]
    hwb = pl.program_id(1)
    a1 = a1_ref[0]

    hw = hwb * HW_BLK + lax.broadcasted_iota(jnp.int32, (HW_BLK, 8), 0)
    iv = (hw // 16).astype(f32) * gsd
    jv = (hw % 16).astype(f32) * gsd
    om8 = 1.0 / (10000.0 ** (
        lax.broadcasted_iota(jnp.int32, (HW_BLK, 8), 1).astype(f32) / 8.0))
    aj = jv * om8
    ai = iv * om8
    sp = jnp.concatenate([
        jnp.zeros((HW_BLK, 96), f32),
        jnp.sin(aj), jnp.cos(aj), jnp.sin(ai), jnp.cos(ai),
    ], axis=1)

    o_ref[0] = x_ref[0] + a1[None, :, :] + sp[:, None, :]


def kernel(per_modality_input_tokens, timestamps, channel_embed, patch_size,
           input_res):
    x = per_modality_input_tokens
    b, h, w, t, cg, D = x.shape
    f32 = jnp.float32
    xr = x.reshape(b, h * w, t * cg, D)

    om16 = 1.0 / (10000.0 ** (jnp.arange(16, dtype=f32) / 16.0))
    targ = jnp.arange(t, dtype=f32)[:, None] * om16[None, :]
    pos32 = jnp.concatenate([jnp.sin(targ), jnp.cos(targ)], axis=1)
    mang = jnp.arange(12, dtype=f32) / f32(12.0 / (2.0 * math.pi))
    mtab = jnp.concatenate([
        jnp.broadcast_to(jnp.sin(mang)[:, None], (12, 16)),
        jnp.broadcast_to(jnp.cos(mang)[:, None], (12, 16)),
        jnp.zeros((12, D - 32), f32),
    ], axis=1)

    ch48 = jnp.tile(channel_embed.astype(f32), (t, 1))
    pos48 = jnp.repeat(pos32, cg, axis=0)
    a1n = jnp.concatenate(
        [ch48, pos48, jnp.zeros((t * cg, 64), f32)], axis=1)

    months = jnp.zeros((b, 16), jnp.int32)
    months = months.at[:, :t].set(timestamps[:, 1, :].astype(jnp.int32))

    mesh = plsc.VectorSubcoreMesh(core_axis_name="c", subcore_axis_name="s")
    a1_full = pl.kernel(
        _sc_table_body, mesh=mesh,
        out_type=jax.ShapeDtypeStruct((b, t * cg, D), f32),
        scratch_types=[
            pltpu.VMEM((t * cg, D), f32),
            pltpu.VMEM((16,), jnp.int32),
            pltpu.VMEM((16, D), f32),
            pltpu.SemaphoreType.DMA,
        ],
    )(a1n, months, mtab)

    gsd = (jnp.asarray(input_res).astype(f32)
           * jnp.asarray(patch_size).astype(f32) / BASE_GSD).reshape(1, 1)
    out = pl.pallas_call(
        _tc_dense_body,
        grid=(b, (h * w) // HW_BLK),
        in_specs=[
            pl.BlockSpec(memory_space=pltpu.SMEM),
            pl.BlockSpec((1, t * cg, D), lambda bi, hi: (bi, 0, 0)),
            pl.BlockSpec((1, HW_BLK, t * cg, D), lambda bi, hi: (bi, hi, 0, 0)),
        ],
        out_specs=pl.BlockSpec((1, HW_BLK, t * cg, D),
                               lambda bi, hi: (bi, hi, 0, 0)),
        out_shape=jax.ShapeDtypeStruct(xr.shape, xr.dtype),
        compiler_params=pltpu.CompilerParams(
            dimension_semantics=("parallel", "parallel")),
    )(gsd, a1_full, xr)
    return out.reshape(b, h, w, t, cg, D)


# speedup vs baseline: 1.1935x; 1.0375x over previous
import math

import jax
import jax.numpy as jnp
from jax import lax
from jax.experimental import pallas as pl
from jax.experimental.pallas import tpu as pltpu
from jax.experimental.pallas import tpu_sc as plsc

BASE_GSD = 10.0
HW_BLK = 256


def _sc_table_body(a1n, months, mtab, out, a1_v, mons_v, mrows_v, si):
    wid = lax.axis_index("s") * 2 + lax.axis_index("c")
    nb = out.shape[0]

    @pl.when(wid < nb)
    def _():
        b = wid
        pltpu.sync_copy(a1n, a1_v)
        pltpu.sync_copy(months.at[b], mons_v)
        pltpu.async_copy(mtab.at[mons_v], mrows_v, si).wait()
        for t in range(12):
            m0 = mrows_v[t, pl.ds(0, 16)]
            m1 = mrows_v[t, pl.ds(16, 16)]
            for c4 in range(4):
                a1_v[4 * t + c4, pl.ds(64, 16)] = m0
                a1_v[4 * t + c4, pl.ds(80, 16)] = m1
        pltpu.sync_copy(a1_v, out.at[b])


def _tc_dense_body(gsd_ref, a1_ref, x_ref, o_ref):
    f32 = jnp.float32
    gsd = gsd_ref[0, 0]
    hwb = pl.program_id(1)
    a1 = a1_ref[0]

    hw = hwb * HW_BLK + lax.broadcasted_iota(jnp.int32, (HW_BLK, 8), 0)
    iv = (hw // 16).astype(f32) * gsd
    jv = (hw % 16).astype(f32) * gsd
    om8 = 1.0 / (10000.0 ** (
        lax.broadcasted_iota(jnp.int32, (HW_BLK, 8), 1).astype(f32) / 8.0))
    aj = jv * om8
    ai = iv * om8
    sp = jnp.concatenate([
        jnp.zeros((HW_BLK, 96), f32),
        jnp.sin(aj), jnp.cos(aj), jnp.sin(ai), jnp.cos(ai),
    ], axis=1)

    o_ref[0] = x_ref[0] + a1[None, :, :] + sp[:, None, :]


def kernel(per_modality_input_tokens, timestamps, channel_embed, patch_size,
           input_res):
    x = per_modality_input_tokens
    b, h, w, t, cg, D = x.shape
    f32 = jnp.float32
    xr = x.reshape(b, h * w, t * cg, D)

    om16 = 1.0 / (10000.0 ** (jnp.arange(16, dtype=f32) / 16.0))
    targ = jnp.arange(t, dtype=f32)[:, None] * om16[None, :]
    pos32 = jnp.concatenate([jnp.sin(targ), jnp.cos(targ)], axis=1)
    mang = jnp.arange(12, dtype=f32) / f32(12.0 / (2.0 * math.pi))
    mtab = jnp.concatenate([
        jnp.broadcast_to(jnp.sin(mang)[:, None], (12, 16)),
        jnp.broadcast_to(jnp.cos(mang)[:, None], (12, 16)),
        jnp.zeros((12, D - 32), f32),
    ], axis=1)

    ch48 = jnp.tile(channel_embed.astype(f32), (t, 1))
    pos48 = jnp.repeat(pos32, cg, axis=0)
    a1n = jnp.concatenate(
        [ch48, pos48, jnp.zeros((t * cg, 64), f32)], axis=1)

    months = jnp.zeros((b, 16), jnp.int32)
    months = months.at[:, :t].set(timestamps[:, 1, :].astype(jnp.int32))

    mesh = plsc.VectorSubcoreMesh(core_axis_name="c", subcore_axis_name="s")
    a1_full = pl.kernel(
        _sc_table_body, mesh=mesh,
        out_type=jax.ShapeDtypeStruct((b, t * cg, D), f32),
        scratch_types=[
            pltpu.VMEM((t * cg, D), f32),
            pltpu.VMEM((16,), jnp.int32),
            pltpu.VMEM((16, D), f32),
            pltpu.SemaphoreType.DMA,
        ],
    )(a1n, months, mtab)

    gsd = (jnp.asarray(input_res).astype(f32)
           * jnp.asarray(patch_size).astype(f32) / BASE_GSD).reshape(1, 1)
    out = pl.pallas_call(
        _tc_dense_body,
        grid=(b, (h * w) // HW_BLK),
        in_specs=[
            pl.BlockSpec(memory_space=pltpu.SMEM),
            pl.BlockSpec((1, t * cg, D), lambda bi, hi: (bi, 0, 0)),
            pl.BlockSpec((1, HW_BLK, t * cg, D), lambda bi, hi: (bi, hi, 0, 0)),
        ],
        out_specs=pl.BlockSpec((1, HW_BLK, t * cg, D),
                               lambda bi, hi: (bi, hi, 0, 0)),
        out_shape=jax.ShapeDtypeStruct(xr.shape, xr.dtype),
        compiler_params=pltpu.CompilerParams(
            dimension_semantics=("parallel", "parallel")),
    )(gsd, a1_full, xr)
    return out.reshape(b, h, w, t, cg, D)


# speedup vs baseline: 1.1976x; 1.0034x over previous
import math

import jax
import jax.numpy as jnp
from jax import lax
from jax.experimental import pallas as pl
from jax.experimental.pallas import tpu as pltpu
from jax.experimental.pallas import tpu_sc as plsc

BASE_GSD = 10.0
HW_BLK = 256
B_BLK = 1


def _sc_table_body(a1n, months, mtab, out, a1_v, mons_v, mrows_v, si):
    wid = lax.axis_index("s") * 2 + lax.axis_index("c")
    nb = out.shape[0]

    @pl.when(wid < nb)
    def _():
        b = wid
        pltpu.sync_copy(a1n, a1_v)
        pltpu.sync_copy(months.at[b], mons_v)
        pltpu.async_copy(mtab.at[mons_v], mrows_v, si).wait()
        for t in range(12):
            m0 = mrows_v[t, pl.ds(0, 16)]
            m1 = mrows_v[t, pl.ds(16, 16)]
            for c4 in range(4):
                a1_v[4 * t + c4, pl.ds(64, 16)] = m0
                a1_v[4 * t + c4, pl.ds(80, 16)] = m1
        pltpu.sync_copy(a1_v, out.at[b])


def _tc_dense_body(gsd_ref, a1_ref, x_ref, o_ref):
    f32 = jnp.float32
    gsd = gsd_ref[0, 0]
    hwb = pl.program_id(1)
    a1 = a1_ref[...]

    hw = hwb * HW_BLK + lax.broadcasted_iota(jnp.int32, (HW_BLK, 8), 0)
    iv = (hw // 16).astype(f32) * gsd
    jv = (hw % 16).astype(f32) * gsd
    om8 = 1.0 / (10000.0 ** (
        lax.broadcasted_iota(jnp.int32, (HW_BLK, 8), 1).astype(f32) / 8.0))
    aj = jv * om8
    ai = iv * om8
    sp = jnp.concatenate([
        jnp.zeros((HW_BLK, 96), f32),
        jnp.sin(aj), jnp.cos(aj), jnp.sin(ai), jnp.cos(ai),
    ], axis=1)

    o_ref[...] = x_ref[...] + a1[:, None, :, :] + sp[None, :, None, :]


def kernel(per_modality_input_tokens, timestamps, channel_embed, patch_size,
           input_res):
    x = per_modality_input_tokens
    b, h, w, t, cg, D = x.shape
    f32 = jnp.float32
    xr = x.reshape(b, h * w, t * cg, D)

    om16 = 1.0 / (10000.0 ** (jnp.arange(16, dtype=f32) / 16.0))
    targ = jnp.arange(t, dtype=f32)[:, None] * om16[None, :]
    pos32 = jnp.concatenate([jnp.sin(targ), jnp.cos(targ)], axis=1)
    mang = jnp.arange(12, dtype=f32) / f32(12.0 / (2.0 * math.pi))
    mtab = jnp.concatenate([
        jnp.broadcast_to(jnp.sin(mang)[:, None], (12, 16)),
        jnp.broadcast_to(jnp.cos(mang)[:, None], (12, 16)),
        jnp.zeros((12, D - 32), f32),
    ], axis=1)

    ch48 = jnp.tile(channel_embed.astype(f32), (t, 1))
    pos48 = jnp.repeat(pos32, cg, axis=0)
    a1n = jnp.concatenate(
        [ch48, pos48, jnp.zeros((t * cg, 64), f32)], axis=1)

    months = jnp.zeros((b, 16), jnp.int32)
    months = months.at[:, :t].set(timestamps[:, 1, :].astype(jnp.int32))

    mesh = plsc.VectorSubcoreMesh(core_axis_name="c", subcore_axis_name="s")
    a1_full = pl.kernel(
        _sc_table_body, mesh=mesh,
        out_type=jax.ShapeDtypeStruct((b, t * cg, D), f32),
        scratch_types=[
            pltpu.VMEM((t * cg, D), f32),
            pltpu.VMEM((16,), jnp.int32),
            pltpu.VMEM((16, D), f32),
            pltpu.SemaphoreType.DMA,
        ],
    )(a1n, months, mtab)

    gsd = (jnp.asarray(input_res).astype(f32)
           * jnp.asarray(patch_size).astype(f32) / BASE_GSD).reshape(1, 1)
    out = pl.pallas_call(
        _tc_dense_body,
        grid=(b // B_BLK, (h * w) // HW_BLK),
        in_specs=[
            pl.BlockSpec(memory_space=pltpu.SMEM),
            pl.BlockSpec((B_BLK, t * cg, D), lambda bi, hi: (bi, 0, 0)),
            pl.BlockSpec((B_BLK, HW_BLK, t * cg, D),
                         lambda bi, hi: (bi, hi, 0, 0)),
        ],
        out_specs=pl.BlockSpec((B_BLK, HW_BLK, t * cg, D),
                               lambda bi, hi: (bi, hi, 0, 0)),
        out_shape=jax.ShapeDtypeStruct(xr.shape, xr.dtype),
        compiler_params=pltpu.CompilerParams(
            dimension_semantics=("parallel", "parallel")),
    )(gsd, a1_full, xr)
    return out.reshape(b, h, w, t, cg, D)


# speedup vs baseline: 1.2237x; 1.0218x over previous
import math

import jax
import jax.numpy as jnp
from jax import lax
from jax.experimental import pallas as pl
from jax.experimental.pallas import tpu as pltpu
from jax.experimental.pallas import tpu_sc as plsc

BASE_GSD = 10.0
HW_BLK = 256
B_BLK = 1


def _sc_gather_body(months, mtab, out, mons_v, mrows_v, si):
    sid = lax.axis_index("s")
    cid = lax.axis_index("c")
    nb = out.shape[0]

    @pl.when((sid < nb) & (cid == 0))
    def _():
        b = sid
        pltpu.sync_copy(months.at[b], mons_v)
        pltpu.async_copy(mtab.at[mons_v], mrows_v, si).wait()
        pltpu.sync_copy(mrows_v, out.at[b])


def _tc_dense_body(gsd_ref, a1n_ref, mrows_ref, x_ref, o_ref):
    f32 = jnp.float32
    gsd = gsd_ref[0, 0]
    hwb = pl.program_id(1)
    t, cg = 12, 4

    m48 = jnp.broadcast_to(
        mrows_ref[0][:t, None, :], (t, cg, 128)).reshape(t * cg, 128)
    a1 = a1n_ref[...] + m48

    hw = hwb * HW_BLK + lax.broadcasted_iota(jnp.int32, (HW_BLK, 8), 0)
    iv = (hw // 16).astype(f32) * gsd
    jv = (hw % 16).astype(f32) * gsd
    om8 = 1.0 / (10000.0 ** (
        lax.broadcasted_iota(jnp.int32, (HW_BLK, 8), 1).astype(f32) / 8.0))
    aj = jv * om8
    ai = iv * om8
    sp = jnp.concatenate([
        jnp.zeros((HW_BLK, 96), f32),
        jnp.sin(aj), jnp.cos(aj), jnp.sin(ai), jnp.cos(ai),
    ], axis=1)

    o_ref[0] = x_ref[0] + a1[None, :, :] + sp[:, None, :]


def kernel(per_modality_input_tokens, timestamps, channel_embed, patch_size,
           input_res):
    x = per_modality_input_tokens
    b, h, w, t, cg, D = x.shape
    f32 = jnp.float32
    xr = x.reshape(b, h * w, t * cg, D)

    om16 = 1.0 / (10000.0 ** (jnp.arange(16, dtype=f32) / 16.0))
    targ = jnp.arange(t, dtype=f32)[:, None] * om16[None, :]
    pos32 = jnp.concatenate([jnp.sin(targ), jnp.cos(targ)], axis=1)
    mang = jnp.arange(12, dtype=f32) / f32(12.0 / (2.0 * math.pi))
    mtab = jnp.concatenate([
        jnp.zeros((12, 64), f32),
        jnp.broadcast_to(jnp.sin(mang)[:, None], (12, 16)),
        jnp.broadcast_to(jnp.cos(mang)[:, None], (12, 16)),
        jnp.zeros((12, D - 96), f32),
    ], axis=1)

    ch48 = jnp.tile(channel_embed.astype(f32), (t, 1))
    pos48 = jnp.repeat(pos32, cg, axis=0)
    a1n = jnp.concatenate(
        [ch48, pos48, jnp.zeros((t * cg, 64), f32)], axis=1)

    months = jnp.zeros((b, 16), jnp.int32)
    months = months.at[:, :t].set(timestamps[:, 1, :].astype(jnp.int32))

    mesh = plsc.VectorSubcoreMesh(core_axis_name="c", subcore_axis_name="s")
    mrows = pl.kernel(
        _sc_gather_body, mesh=mesh,
        out_type=jax.ShapeDtypeStruct((b, 16, D), f32),
        scratch_types=[
            pltpu.VMEM((16,), jnp.int32),
            pltpu.VMEM((16, D), f32),
            pltpu.SemaphoreType.DMA,
        ],
    )(months, mtab)

    gsd = (jnp.asarray(input_res).astype(f32)
           * jnp.asarray(patch_size).astype(f32) / BASE_GSD).reshape(1, 1)
    out = pl.pallas_call(
        _tc_dense_body,
        grid=(b // B_BLK, (h * w) // HW_BLK),
        in_specs=[
            pl.BlockSpec(memory_space=pltpu.SMEM),
            pl.BlockSpec((t * cg, D), lambda bi, hi: (0, 0)),
            pl.BlockSpec((1, 16, D), lambda bi, hi: (bi, 0, 0)),
            pl.BlockSpec((B_BLK, HW_BLK, t * cg, D),
                         lambda bi, hi: (bi, hi, 0, 0)),
        ],
        out_specs=pl.BlockSpec((B_BLK, HW_BLK, t * cg, D),
                               lambda bi, hi: (bi, hi, 0, 0)),
        out_shape=jax.ShapeDtypeStruct(xr.shape, xr.dtype),
        compiler_params=pltpu.CompilerParams(
            dimension_semantics=("parallel", "parallel")),
    )(gsd, a1n, mrows, xr)
    return out.reshape(b, h, w, t, cg, D)
